# Initial kernel scaffold; baseline (speedup 1.0000x reference)
import functools

import jax
import jax.numpy as jnp
from jax import lax
from jax.experimental import pallas as pl
from jax.experimental.pallas import tpu as pltpu
from jax.experimental.pallas import tpu_sc as plsc

F32 = jnp.float32
BF16 = jnp.bfloat16
NEGL = -1e30
H = 128
H4 = 512


def _mm(a, b):
    return jnp.dot(a.astype(BF16), b.astype(BF16), preferred_element_type=F32)


def _tableproj_body(wv_ref, wt_ref, out_ref):
    out_ref[...] = _mm(wv_ref[...], wt_ref[...])


def _project_table(wv, wprojT):
    Vv, Dd = wv.shape
    Hh = wprojT.shape[1]
    blk = 4000
    return pl.pallas_call(
        _tableproj_body,
        grid=(Vv // blk,),
        in_specs=[pl.BlockSpec((blk, Dd), lambda i: (i, 0)),
                  pl.BlockSpec((Dd, Hh), lambda i: (0, 0))],
        out_specs=pl.BlockSpec((blk, Hh), lambda i: (i, 0)),
        out_shape=jax.ShapeDtypeStruct((Vv, Hh), F32),
    )(wv, wprojT)


def _sc_gather(table, idx):
    info = plsc.get_sparse_core_info()
    nc, ns = info.num_cores, info.num_subcores
    nw = nc * ns
    n = idx.shape[0]
    bpw = n // nw
    Hh = table.shape[1]
    mesh = plsc.VectorSubcoreMesh(core_axis_name="c", subcore_axis_name="s")

    @functools.partial(
        pl.kernel, mesh=mesh,
        out_type=jax.ShapeDtypeStruct((n, Hh), F32),
        scratch_types=[pltpu.VMEM((bpw,), jnp.int32),
                       pltpu.VMEM((bpw, Hh), F32),
                       pltpu.SemaphoreType.DMA],
    )
    def gk(table_hbm, idx_hbm, out_hbm, idx_v, rows_v, sem):
        wid = lax.axis_index("s") * nc + lax.axis_index("c")
        base = wid * bpw
        pltpu.sync_copy(idx_hbm.at[pl.ds(base, bpw)], idx_v)
        pltpu.async_copy(table_hbm.at[idx_v], rows_v, sem).wait()
        pltpu.sync_copy(rows_v, out_hbm.at[pl.ds(base, bpw)])

    return gk(table, idx)


def _embed_body(e_ref, bert_ref, wg1, bg1, wt1, bt1, wg2, bg2, wt2, bt2,
                wb, bb, out_ref):
    x = e_ref[...]
    for wg, bg, wt, bt in ((wg1, bg1, wt1, bt1), (wg2, bg2, wt2, bt2)):
        g = jax.nn.sigmoid(_mm(x, wg[...]) + bg[...])
        t = jnp.maximum(_mm(x, wt[...]) + bt[...], 0.0)
        x = g * t + (1.0 - g) * x
    bh = jnp.maximum(_mm(bert_ref[...], wb[...]) + bb[...], 0.0)
    out_ref[...] = x * (1.0 + bh)


def _embed_hw(e_all, bert_all, p):
    n, Hh = e_all.shape
    Db = bert_all.shape[1]
    blk = 1440
    w = lambda k: p[k].T
    b = lambda k: p[k].reshape(1, -1)
    args = (w('Wg1'), b('bg1'), w('Wt1'), b('bt1'),
            w('Wg2'), b('bg2'), w('Wt2'), b('bt2'),
            w('Wbert'), b('bbert'))
    return pl.pallas_call(
        _embed_body,
        grid=(n // blk,),
        in_specs=[pl.BlockSpec((blk, Hh), lambda i: (i, 0)),
                  pl.BlockSpec((blk, Db), lambda i: (i, 0))]
                 + [pl.BlockSpec(a.shape, lambda i: (0, 0)) for a in args],
        out_specs=pl.BlockSpec((blk, Hh), lambda i: (i, 0)),
        out_shape=jax.ShapeDtypeStruct((n, Hh), F32),
    )(e_all, bert_all, *args)


def _bilstm_body(TB, Bb, Din, xf_ref, xb_ref, mf_ref, mb_ref,
                 wf_ref, bf_ref, wb_ref, bb_ref, wc_ref,
                 outf_ref, outb_ref, xpf_s, xpb_s, hc_s, wcb_s):
    j = pl.program_id(0)

    @pl.when(j == 0)
    def _():
        hc_s[...] = jnp.zeros_like(hc_s)

    wcb_s[...] = wc_ref[...].astype(BF16)

    xpf = _mm(xf_ref[...].reshape(TB * Bb, Din), wf_ref[...]) + bf_ref[...]
    xpf_s[...] = xpf.reshape(TB, Bb, H4)
    xpb = _mm(xb_ref[...].reshape(TB * Bb, Din), wb_ref[...]) + bb_ref[...]
    xpb_s[...] = xpb.reshape(TB, Bb, H4)

    def step(k, _):
        kk = TB - 1 - k
        h = hc_s[0]
        c = hc_s[1]
        z64 = jnp.dot(h.astype(BF16), wcb_s[...], preferred_element_type=F32)
        zf = z64[0:Bb, 0:H4] + xpf_s[k]
        zb = z64[Bb:2 * Bb, H4:2 * H4] + xpb_s[kk]
        z = jnp.concatenate([zf, zb], axis=0)
        i_ = jax.nn.sigmoid(z[:, 0:H])
        f_ = jax.nn.sigmoid(z[:, H:2 * H])
        g_ = jnp.tanh(z[:, 2 * H:3 * H])
        o_ = jax.nn.sigmoid(z[:, 3 * H:4 * H])
        c_new = f_ * c + i_ * g_
        h_new = o_ * jnp.tanh(c_new)
        hc_s[0] = h_new
        hc_s[1] = c_new
        outf_ref[k] = h_new[0:Bb]
        outb_ref[kk] = h_new[Bb:2 * Bb]
        return 0

    lax.fori_loop(0, TB, step, 0)
    outf_ref[...] = outf_ref[...] * mf_ref[...][:, :, None]
    outb_ref[...] = outb_ref[...] * mb_ref[...][:, :, None]


def _bilstm(x_t, mask_t, p, TB):
    T, Bb, Din = x_t.shape
    G = T // TB
    wf = p['Wih_f'].T
    wb = p['Wih_b'].T
    bf = p['b_f'].reshape(1, -1)
    bb = p['b_b'].reshape(1, -1)
    wc = jnp.concatenate([p['Whh_f'].T, p['Whh_b'].T], axis=1)
    body = functools.partial(_bilstm_body, TB, Bb, Din)
    outf, outb = pl.pallas_call(
        body,
        grid=(G,),
        in_specs=[
            pl.BlockSpec((TB, Bb, Din), lambda j: (j, 0, 0)),
            pl.BlockSpec((TB, Bb, Din), lambda j, G=G: (G - 1 - j, 0, 0)),
            pl.BlockSpec((TB, Bb), lambda j: (j, 0)),
            pl.BlockSpec((TB, Bb), lambda j, G=G: (G - 1 - j, 0)),
            pl.BlockSpec((Din, H4), lambda j: (0, 0)),
            pl.BlockSpec((1, H4), lambda j: (0, 0)),
            pl.BlockSpec((Din, H4), lambda j: (0, 0)),
            pl.BlockSpec((1, H4), lambda j: (0, 0)),
            pl.BlockSpec((H, 2 * H4), lambda j: (0, 0)),
        ],
        out_specs=[
            pl.BlockSpec((TB, Bb, H), lambda j: (j, 0, 0)),
            pl.BlockSpec((TB, Bb, H), lambda j, G=G: (G - 1 - j, 0, 0)),
        ],
        out_shape=[jax.ShapeDtypeStruct((T, Bb, H), F32),
                   jax.ShapeDtypeStruct((T, Bb, H), F32)],
        scratch_shapes=[pltpu.VMEM((TB, Bb, H4), F32),
                        pltpu.VMEM((TB, Bb, H4), F32),
                        pltpu.VMEM((2, 2 * Bb, H), F32),
                        pltpu.VMEM((H, 2 * H4), BF16)],
    )(x_t, x_t, mask_t, mask_t, wf, bf, wb, bb, wc)
    return outf, outb


def _att_body(GB, Tc, Tq, c_ref, q_ref, cm_ref, qm_ref, wq_ref, bq_ref, out_ref):
    c = jnp.transpose(c_ref[...], (1, 0, 2))
    q = jnp.transpose(q_ref[...], (1, 0, 2))
    cm = jnp.transpose(cm_ref[...], (1, 0))
    qm = jnp.transpose(qm_ref[...], (1, 0))
    D2 = c.shape[2]
    qp = jnp.tanh(_mm(q.reshape(GB * Tq, D2), wq_ref[...]).reshape(GB, Tq, D2)
                  + bq_ref[...])
    Lg = lax.dot_general(c.astype(BF16), qp.astype(BF16),
                         (((2,), (2,)), ((0,), (0,))),
                         preferred_element_type=F32)
    La = jnp.where(qm[:, None, :] > 0, Lg, NEGL)
    A = jax.nn.softmax(La, axis=2)
    Lb = jnp.where(cm[:, :, None] > 0, Lg, NEGL)
    Bm = jax.nn.softmax(Lb, axis=1)
    c2q = lax.dot_general(A.astype(BF16), qp.astype(BF16),
                          (((2,), (1,)), ((0,), (0,))),
                          preferred_element_type=F32)
    q2c = lax.dot_general(Bm.astype(BF16), c.astype(BF16),
                          (((1,), (1,)), ((0,), (0,))),
                          preferred_element_type=F32)
    coatt = lax.dot_general(A.astype(BF16), q2c.astype(BF16),
                            (((2,), (1,)), ((0,), (0,))),
                            preferred_element_type=F32)
    c2q_t = jnp.transpose(c2q, (1, 0, 2))
    coatt_t = jnp.transpose(coatt, (1, 0, 2))
    cv = c_ref[...]
    out_ref[:, :, 0:D2] = cv
    out_ref[:, :, D2:2 * D2] = c2q_t
    out_ref[:, :, 2 * D2:3 * D2] = cv * c2q_t
    out_ref[:, :, 3 * D2:4 * D2] = cv * coatt_t


def _attention(c_enc, q_enc, cm_t, qm_t, p):
    Tc, Bb, D2 = c_enc.shape
    Tq = q_enc.shape[0]
    GB = 8
    wq = p['Wq'].T
    bq = p['bq'].reshape(1, 1, -1)
    body = functools.partial(_att_body, GB, Tc, Tq)
    return pl.pallas_call(
        body,
        grid=(Bb // GB,),
        in_specs=[
            pl.BlockSpec((Tc, GB, D2), lambda i: (0, i, 0)),
            pl.BlockSpec((Tq, GB, D2), lambda i: (0, i, 0)),
            pl.BlockSpec((Tc, GB), lambda i: (0, i)),
            pl.BlockSpec((Tq, GB), lambda i: (0, i)),
            pl.BlockSpec((D2, D2), lambda i: (0, 0)),
            pl.BlockSpec((1, 1, D2), lambda i: (0, 0, 0)),
        ],
        out_specs=pl.BlockSpec((Tc, GB, 4 * D2), lambda i: (0, i, 0)),
        out_shape=jax.ShapeDtypeStruct((Tc, Bb, 4 * D2), F32),
    )(c_enc, q_enc, cm_t, qm_t, wq, bq)


def _logits_body(att_ref, mod_ref, mod2_ref, wa1, wm1, wa2, wm2,
                 l1_ref, l2_ref):
    att = att_ref[...]
    l1_ref[...] = (jnp.sum(att * wa1[...], axis=2)
                   + jnp.sum(mod_ref[...] * wm1[...], axis=2))
    l2_ref[...] = (jnp.sum(att * wa2[...], axis=2)
                   + jnp.sum(mod2_ref[...] * wm2[...], axis=2))


def _logits(att, mod, mod2, p):
    Tc, Bb, D8 = att.shape
    D2 = mod.shape[2]
    TB = 80
    v = lambda k: p[k].reshape(1, 1, -1)
    return pl.pallas_call(
        _logits_body,
        grid=(Tc // TB,),
        in_specs=[
            pl.BlockSpec((TB, Bb, D8), lambda i: (i, 0, 0)),
            pl.BlockSpec((TB, Bb, D2), lambda i: (i, 0, 0)),
            pl.BlockSpec((TB, Bb, D2), lambda i: (i, 0, 0)),
            pl.BlockSpec((1, 1, D8), lambda i: (0, 0, 0)),
            pl.BlockSpec((1, 1, D2), lambda i: (0, 0, 0)),
            pl.BlockSpec((1, 1, D8), lambda i: (0, 0, 0)),
            pl.BlockSpec((1, 1, D2), lambda i: (0, 0, 0)),
        ],
        out_specs=[pl.BlockSpec((TB, Bb), lambda i: (i, 0)),
                   pl.BlockSpec((TB, Bb), lambda i: (i, 0))],
        out_shape=[jax.ShapeDtypeStruct((Tc, Bb), F32),
                   jax.ShapeDtypeStruct((Tc, Bb), F32)],
    )(att, mod, mod2, v('Watt1'), v('Wmod1'), v('Watt2'), v('Wmod2'))


def _lsm_body(l1_ref, l2_ref, m_ref, o1_ref, o2_ref):
    m = m_ref[...] > 0
    for lr, orr in ((l1_ref, o1_ref), (l2_ref, o2_ref)):
        x = jnp.where(m, lr[...], NEGL)
        mx = jnp.max(x, axis=0, keepdims=True)
        e = jnp.exp(x - mx)
        s = jnp.sum(e, axis=0, keepdims=True)
        orr[...] = x - mx - jnp.log(s)


def _logsoftmax(l1, l2, cm_t):
    Tc, Bb = l1.shape
    return pl.pallas_call(
        _lsm_body,
        out_shape=[jax.ShapeDtypeStruct((Tc, Bb), F32),
                   jax.ShapeDtypeStruct((Tc, Bb), F32)],
    )(l1, l2, cm_t)


def kernel(cw_idxs, qw_idxs, bert_embeddings, max_context_len,
           max_question_len, device, params, word_vectors):
    p = params
    Bb, mc = cw_idxs.shape
    mq = qw_idxs.shape[1]
    cw = cw_idxs.astype(jnp.int32)
    qw = qw_idxs.astype(jnp.int32)
    c_mask = ((cw != 0) & (jnp.arange(mc) < max_context_len)[None, :]).astype(F32)
    q_mask = ((qw != 0) & (jnp.arange(mq) < max_question_len)[None, :]).astype(F32)
    cm_t = c_mask.T
    qm_t = q_mask.T

    idx_t = jnp.concatenate([cw, qw], axis=1).T.reshape(-1)
    ntok = idx_t.shape[0]
    npad = ((ntok + 255) // 256) * 256
    idx_pad = jnp.zeros((npad,), jnp.int32).at[:ntok].set(idx_t)

    tp = _project_table(word_vectors, p['Wproj'].T)
    e_all = _sc_gather(tp, idx_pad)[:ntok]

    bert_t = jnp.transpose(bert_embeddings, (1, 0, 2)).reshape(ntok, -1)
    x_all = _embed_hw(e_all, bert_t, p).reshape(mc + mq, Bb, H)
    c_emb = x_all[:mc]
    q_emb = x_all[mc:]

    cf, cb = _bilstm(c_emb, cm_t, p['enc'], TB=40)
    qf, qb = _bilstm(q_emb, qm_t, p['enc'], TB=mq)
    c_enc = jnp.concatenate([cf, cb], axis=2)
    q_enc = jnp.concatenate([qf, qb], axis=2)

    att = _attention(c_enc, q_enc, cm_t, qm_t, p)

    m1f, m1b = _bilstm(att, cm_t, p['mod1'], TB=40)
    mod = jnp.concatenate([m1f, m1b], axis=2)
    m2f, m2b = _bilstm(mod, cm_t, p['mod2'], TB=40)
    mod = jnp.concatenate([m2f, m2b], axis=2)
    mof, mob = _bilstm(mod, cm_t, p['out_rnn'], TB=40)
    mod_2 = jnp.concatenate([mof, mob], axis=2)

    l1, l2 = _logits(att, mod, mod_2, p)
    lp1, lp2 = _logsoftmax(l1, l2, cm_t)
    return lp1.T, lp2.T


# speedup vs baseline: 13.2733x; 13.2733x over previous
import functools

import jax
import jax.numpy as jnp
from jax import lax
from jax.experimental import pallas as pl
from jax.experimental.pallas import tpu as pltpu
from jax.experimental.pallas import tpu_sc as plsc

F32 = jnp.float32
BF16 = jnp.bfloat16
NEGL = -1e30
H = 128
H4 = 512


def _mm(a, b):
    return jnp.dot(a.astype(BF16), b.astype(BF16), preferred_element_type=F32)


def _tableproj_body(wv_ref, wt_ref, out_ref):
    out_ref[...] = _mm(wv_ref[...], wt_ref[...])


def _project_table(wv, wprojT):
    Vv, Dd = wv.shape
    Hh = wprojT.shape[1]
    blk = 4000
    return pl.pallas_call(
        _tableproj_body,
        grid=(Vv // blk,),
        in_specs=[pl.BlockSpec((blk, Dd), lambda i: (i, 0)),
                  pl.BlockSpec((Dd, Hh), lambda i: (0, 0))],
        out_specs=pl.BlockSpec((blk, Hh), lambda i: (i, 0)),
        out_shape=jax.ShapeDtypeStruct((Vv, Hh), F32),
    )(wv, wprojT)


def _sc_gather(table, idx):
    info = plsc.get_sparse_core_info()
    nc, ns = info.num_cores, info.num_subcores
    nw = nc * ns
    n = idx.shape[0]
    bpw = n // nw
    Hh = table.shape[1]
    mesh = plsc.VectorSubcoreMesh(core_axis_name="c", subcore_axis_name="s")

    @functools.partial(
        pl.kernel, mesh=mesh,
        out_type=jax.ShapeDtypeStruct((n, Hh), F32),
        scratch_types=[pltpu.VMEM((bpw,), jnp.int32),
                       pltpu.VMEM((bpw, Hh), F32),
                       pltpu.SemaphoreType.DMA],
    )
    def gk(table_hbm, idx_hbm, out_hbm, idx_v, rows_v, sem):
        wid = lax.axis_index("s") * nc + lax.axis_index("c")
        base = wid * bpw
        pltpu.sync_copy(idx_hbm.at[pl.ds(base, bpw)], idx_v)
        pltpu.async_copy(table_hbm.at[idx_v], rows_v, sem).wait()
        pltpu.sync_copy(rows_v, out_hbm.at[pl.ds(base, bpw)])

    return gk(table, idx)


def _embed_body(e_ref, bert_ref, wg1, bg1, wt1, bt1, wg2, bg2, wt2, bt2,
                wb, bb, out_ref):
    x = e_ref[...]
    for wg, bg, wt, bt in ((wg1, bg1, wt1, bt1), (wg2, bg2, wt2, bt2)):
        g = jax.nn.sigmoid(_mm(x, wg[...]) + bg[...])
        t = jnp.maximum(_mm(x, wt[...]) + bt[...], 0.0)
        x = g * t + (1.0 - g) * x
    bh = jnp.maximum(_mm(bert_ref[...], wb[...]) + bb[...], 0.0)
    out_ref[...] = x * (1.0 + bh)


def _embed_hw(e_all, bert_all, p):
    n, Hh = e_all.shape
    Db = bert_all.shape[1]
    blk = 1440
    w = lambda k: p[k].T
    b = lambda k: p[k].reshape(1, -1)
    args = (w('Wg1'), b('bg1'), w('Wt1'), b('bt1'),
            w('Wg2'), b('bg2'), w('Wt2'), b('bt2'),
            w('Wbert'), b('bbert'))
    return pl.pallas_call(
        _embed_body,
        grid=(n // blk,),
        in_specs=[pl.BlockSpec((blk, Hh), lambda i: (i, 0)),
                  pl.BlockSpec((blk, Db), lambda i: (i, 0))]
                 + [pl.BlockSpec(a.shape, lambda i: (0, 0)) for a in args],
        out_specs=pl.BlockSpec((blk, Hh), lambda i: (i, 0)),
        out_shape=jax.ShapeDtypeStruct((n, Hh), F32),
    )(e_all, bert_all, *args)


def _bilstm_body(TB, Bb, Din, xf_ref, xb_ref, mf_ref, mb_ref,
                 wf_ref, bf_ref, wb_ref, bb_ref, wc_ref,
                 outf_ref, outb_ref, xpf_s, xpb_s, hc_s, wcb_s):
    j = pl.program_id(0)

    @pl.when(j == 0)
    def _():
        hc_s[...] = jnp.zeros_like(hc_s)

    wcb_s[...] = wc_ref[...].astype(BF16)

    xpf = _mm(xf_ref[...].reshape(TB * Bb, Din), wf_ref[...]) + bf_ref[...]
    xpf_s[...] = xpf.reshape(TB, Bb, H4)
    xpb = _mm(xb_ref[...].reshape(TB * Bb, Din), wb_ref[...]) + bb_ref[...]
    xpb_s[...] = xpb.reshape(TB, Bb, H4)

    def step(k, _):
        kk = TB - 1 - k
        h = hc_s[0]
        c = hc_s[1]
        z64 = jnp.dot(h.astype(BF16), wcb_s[...], preferred_element_type=F32)
        zf = z64[0:Bb, 0:H4] + xpf_s[k]
        zb = z64[Bb:2 * Bb, H4:2 * H4] + xpb_s[kk]
        z = jnp.concatenate([zf, zb], axis=0)
        i_ = jax.nn.sigmoid(z[:, 0:H])
        f_ = jax.nn.sigmoid(z[:, H:2 * H])
        g_ = jnp.tanh(z[:, 2 * H:3 * H])
        o_ = jax.nn.sigmoid(z[:, 3 * H:4 * H])
        c_new = f_ * c + i_ * g_
        h_new = o_ * jnp.tanh(c_new)
        hc_s[0] = h_new
        hc_s[1] = c_new
        outf_ref[k] = h_new[0:Bb]
        outb_ref[kk] = h_new[Bb:2 * Bb]
        return 0

    lax.fori_loop(0, TB, step, 0)
    outf_ref[...] = outf_ref[...] * mf_ref[...][:, :, None]
    outb_ref[...] = outb_ref[...] * mb_ref[...][:, :, None]


def _bilstm(x_t, mask_t, p, TB):
    T, Bb, Din = x_t.shape
    G = T // TB
    wf = p['Wih_f'].T
    wb = p['Wih_b'].T
    bf = p['b_f'].reshape(1, -1)
    bb = p['b_b'].reshape(1, -1)
    wc = jnp.concatenate([p['Whh_f'].T, p['Whh_b'].T], axis=1)
    body = functools.partial(_bilstm_body, TB, Bb, Din)
    outf, outb = pl.pallas_call(
        body,
        grid=(G,),
        in_specs=[
            pl.BlockSpec((TB, Bb, Din), lambda j: (j, 0, 0)),
            pl.BlockSpec((TB, Bb, Din), lambda j, G=G: (G - 1 - j, 0, 0)),
            pl.BlockSpec((TB, Bb), lambda j: (j, 0)),
            pl.BlockSpec((TB, Bb), lambda j, G=G: (G - 1 - j, 0)),
            pl.BlockSpec((Din, H4), lambda j: (0, 0)),
            pl.BlockSpec((1, H4), lambda j: (0, 0)),
            pl.BlockSpec((Din, H4), lambda j: (0, 0)),
            pl.BlockSpec((1, H4), lambda j: (0, 0)),
            pl.BlockSpec((H, 2 * H4), lambda j: (0, 0)),
        ],
        out_specs=[
            pl.BlockSpec((TB, Bb, H), lambda j: (j, 0, 0)),
            pl.BlockSpec((TB, Bb, H), lambda j, G=G: (G - 1 - j, 0, 0)),
        ],
        out_shape=[jax.ShapeDtypeStruct((T, Bb, H), F32),
                   jax.ShapeDtypeStruct((T, Bb, H), F32)],
        scratch_shapes=[pltpu.VMEM((TB, Bb, H4), F32),
                        pltpu.VMEM((TB, Bb, H4), F32),
                        pltpu.VMEM((2, 2 * Bb, H), F32),
                        pltpu.VMEM((H, 2 * H4), BF16)],
    )(x_t, x_t, mask_t, mask_t, wf, bf, wb, bb, wc)
    return outf, outb


def _att_body(GB, Tc, Tq, c_ref, q_ref, cm_ref, qm_ref, wq_ref, bq_ref, out_ref):
    c = jnp.transpose(c_ref[...], (1, 0, 2))
    q = jnp.transpose(q_ref[...], (1, 0, 2))
    cm = cm_ref[0]
    qm = qm_ref[0]
    D2 = c.shape[2]
    qp = jnp.tanh(_mm(q.reshape(GB * Tq, D2), wq_ref[...]).reshape(GB, Tq, D2)
                  + bq_ref[...])
    Lg = lax.dot_general(c.astype(BF16), qp.astype(BF16),
                         (((2,), (2,)), ((0,), (0,))),
                         preferred_element_type=F32)
    La = jnp.where(qm[:, None, :] > 0, Lg, NEGL)
    A = jax.nn.softmax(La, axis=2)
    Lb = jnp.where(cm[:, :, None] > 0, Lg, NEGL)
    Bm = jax.nn.softmax(Lb, axis=1)
    c2q = lax.dot_general(A.astype(BF16), qp.astype(BF16),
                          (((2,), (1,)), ((0,), (0,))),
                          preferred_element_type=F32)
    q2c = lax.dot_general(Bm.astype(BF16), c.astype(BF16),
                          (((1,), (1,)), ((0,), (0,))),
                          preferred_element_type=F32)
    coatt = lax.dot_general(A.astype(BF16), q2c.astype(BF16),
                            (((2,), (1,)), ((0,), (0,))),
                            preferred_element_type=F32)
    c2q_t = jnp.transpose(c2q, (1, 0, 2))
    coatt_t = jnp.transpose(coatt, (1, 0, 2))
    cv = c_ref[...]
    out_ref[:, :, 0:D2] = cv
    out_ref[:, :, D2:2 * D2] = c2q_t
    out_ref[:, :, 2 * D2:3 * D2] = cv * c2q_t
    out_ref[:, :, 3 * D2:4 * D2] = cv * coatt_t


def _attention(c_enc, q_enc, cm_b, qm_b, p):
    Tc, Bb, D2 = c_enc.shape
    Tq = q_enc.shape[0]
    GB = 8
    wq = p['Wq'].T
    bq = p['bq'].reshape(1, 1, -1)
    cm3 = cm_b.reshape(Bb // GB, GB, Tc)
    qm3 = qm_b.reshape(Bb // GB, GB, Tq)
    body = functools.partial(_att_body, GB, Tc, Tq)
    return pl.pallas_call(
        body,
        grid=(Bb // GB,),
        in_specs=[
            pl.BlockSpec((Tc, GB, D2), lambda i: (0, i, 0)),
            pl.BlockSpec((Tq, GB, D2), lambda i: (0, i, 0)),
            pl.BlockSpec((1, GB, Tc), lambda i: (i, 0, 0)),
            pl.BlockSpec((1, GB, Tq), lambda i: (i, 0, 0)),
            pl.BlockSpec((D2, D2), lambda i: (0, 0)),
            pl.BlockSpec((1, 1, D2), lambda i: (0, 0, 0)),
        ],
        out_specs=pl.BlockSpec((Tc, GB, 4 * D2), lambda i: (0, i, 0)),
        out_shape=jax.ShapeDtypeStruct((Tc, Bb, 4 * D2), F32),
    )(c_enc, q_enc, cm3, qm3, wq, bq)


def _logits_body(att_ref, mod_ref, mod2_ref, wa1, wm1, wa2, wm2,
                 l1_ref, l2_ref):
    att = att_ref[...]
    l1_ref[...] = (jnp.sum(att * wa1[...], axis=2)
                   + jnp.sum(mod_ref[...] * wm1[...], axis=2))
    l2_ref[...] = (jnp.sum(att * wa2[...], axis=2)
                   + jnp.sum(mod2_ref[...] * wm2[...], axis=2))


def _logits(att, mod, mod2, p):
    Tc, Bb, D8 = att.shape
    D2 = mod.shape[2]
    TB = 80
    v = lambda k: p[k].reshape(1, 1, -1)
    return pl.pallas_call(
        _logits_body,
        grid=(Tc // TB,),
        in_specs=[
            pl.BlockSpec((TB, Bb, D8), lambda i: (i, 0, 0)),
            pl.BlockSpec((TB, Bb, D2), lambda i: (i, 0, 0)),
            pl.BlockSpec((TB, Bb, D2), lambda i: (i, 0, 0)),
            pl.BlockSpec((1, 1, D8), lambda i: (0, 0, 0)),
            pl.BlockSpec((1, 1, D2), lambda i: (0, 0, 0)),
            pl.BlockSpec((1, 1, D8), lambda i: (0, 0, 0)),
            pl.BlockSpec((1, 1, D2), lambda i: (0, 0, 0)),
        ],
        out_specs=[pl.BlockSpec((TB, Bb), lambda i: (i, 0)),
                   pl.BlockSpec((TB, Bb), lambda i: (i, 0))],
        out_shape=[jax.ShapeDtypeStruct((Tc, Bb), F32),
                   jax.ShapeDtypeStruct((Tc, Bb), F32)],
    )(att, mod, mod2, v('Watt1'), v('Wmod1'), v('Watt2'), v('Wmod2'))


def _lsm_body(l1_ref, l2_ref, m_ref, o1_ref, o2_ref):
    m = m_ref[...] > 0
    for lr, orr in ((l1_ref, o1_ref), (l2_ref, o2_ref)):
        x = jnp.where(m, lr[...], NEGL)
        mx = jnp.max(x, axis=0, keepdims=True)
        e = jnp.exp(x - mx)
        s = jnp.sum(e, axis=0, keepdims=True)
        orr[...] = x - mx - jnp.log(s)


def _logsoftmax(l1, l2, cm_t):
    Tc, Bb = l1.shape
    return pl.pallas_call(
        _lsm_body,
        out_shape=[jax.ShapeDtypeStruct((Tc, Bb), F32),
                   jax.ShapeDtypeStruct((Tc, Bb), F32)],
    )(l1, l2, cm_t)


def kernel(cw_idxs, qw_idxs, bert_embeddings, max_context_len,
           max_question_len, device, params, word_vectors):
    p = params
    Bb, mc = cw_idxs.shape
    mq = qw_idxs.shape[1]
    cw = cw_idxs.astype(jnp.int32)
    qw = qw_idxs.astype(jnp.int32)
    c_mask = ((cw != 0) & (jnp.arange(mc) < max_context_len)[None, :]).astype(F32)
    q_mask = ((qw != 0) & (jnp.arange(mq) < max_question_len)[None, :]).astype(F32)
    cm_t = c_mask.T
    qm_t = q_mask.T

    idx_t = jnp.concatenate([cw, qw], axis=1).T.reshape(-1)
    ntok = idx_t.shape[0]
    npad = ((ntok + 255) // 256) * 256
    idx_pad = jnp.zeros((npad,), jnp.int32).at[:ntok].set(idx_t)

    tp = _project_table(word_vectors, p['Wproj'].T)
    e_all = _sc_gather(tp, idx_pad)[:ntok]

    bert_t = jnp.transpose(bert_embeddings, (1, 0, 2)).reshape(ntok, -1)
    x_all = _embed_hw(e_all, bert_t, p).reshape(mc + mq, Bb, H)
    c_emb = x_all[:mc]
    q_emb = x_all[mc:]

    cf, cb = _bilstm(c_emb, cm_t, p['enc'], TB=40)
    qf, qb = _bilstm(q_emb, qm_t, p['enc'], TB=mq)
    c_enc = jnp.concatenate([cf, cb], axis=2)
    q_enc = jnp.concatenate([qf, qb], axis=2)

    att = _attention(c_enc, q_enc, c_mask, q_mask, p)

    m1f, m1b = _bilstm(att, cm_t, p['mod1'], TB=40)
    mod = jnp.concatenate([m1f, m1b], axis=2)
    m2f, m2b = _bilstm(mod, cm_t, p['mod2'], TB=40)
    mod = jnp.concatenate([m2f, m2b], axis=2)
    mof, mob = _bilstm(mod, cm_t, p['out_rnn'], TB=40)
    mod_2 = jnp.concatenate([mof, mob], axis=2)

    l1, l2 = _logits(att, mod, mod_2, p)
    lp1, lp2 = _logsoftmax(l1, l2, cm_t)
    return lp1.T, lp2.T
